# Initial kernel scaffold; baseline (speedup 1.0000x reference)
import jax
import jax.numpy as jnp
from jax.experimental import pallas as pl


def kernel(x, U_w, U_b, V_w, V_b, bn_gamma, bn_beta):
    raise NotImplementedError("write your pallas kernel here")


# speedup vs baseline: 6.3771x; 6.3771x over previous
import jax
import jax.numpy as jnp
from jax.experimental import pallas as pl

_EPS = 1e-5
_K = 4


def _phase1_kernel(x_ref, uw_ref, ub_ref, vw_ref, vb_ref, y_ref, stats_ref):
    f32 = jnp.float32
    xt = x_ref[0]
    sim = jax.lax.dot_general(
        xt, xt, (((2,), (2,)), ((0,), (0,))), preferred_element_type=f32)
    m = jnp.max(sim, axis=-1, keepdims=True)
    cnt = jnp.sum((sim == m).astype(f32), axis=-1, keepdims=True)
    thr = m
    for _ in range(_K - 1):
        nm = jnp.max(jnp.where(sim < thr, sim, -jnp.inf), axis=-1, keepdims=True)
        c2 = jnp.sum((sim == nm).astype(f32), axis=-1, keepdims=True)
        need = cnt < _K
        thr = jnp.where(need, nm, thr)
        cnt = jnp.where(need, cnt + c2, cnt)
    adj = (sim >= thr).astype(f32)
    deg = jnp.sum(adj, axis=-1, keepdims=True)
    dinv = jax.lax.rsqrt(deg)
    vx = jax.lax.dot_general(
        xt, vw_ref[...], (((2,), (1,)), ((), ())),
        preferred_element_type=f32) + vb_ref[...]
    ux = jax.lax.dot_general(
        xt, uw_ref[...], (((2,), (1,)), ((), ())),
        preferred_element_type=f32) + ub_ref[...]
    agg = jax.lax.dot_general(
        adj, vx * dinv, (((2,), (1,)), ((0,), (0,))),
        preferred_element_type=f32)
    y = agg * dinv + ux
    y_ref[0] = y
    s1 = jnp.sum(jnp.sum(y, axis=-1), axis=0)
    s2 = jnp.sum(jnp.sum(y * y, axis=-1), axis=0)

    @pl.when(pl.program_id(0) == 0)
    def _init():
        stats_ref[...] = jnp.zeros_like(stats_ref)

    stats_ref[0, :] += s1
    stats_ref[1, :] += s2


def _phase2_kernel(x_ref, y_ref, sc_ref, sh_ref, o_ref):
    xt = x_ref[0]
    h = y_ref[0] * sc_ref[0] + sh_ref[0]
    o_ref[0] = jnp.maximum(xt + h, 0.0)


def kernel(x, U_w, U_b, V_w, V_b, bn_gamma, bn_beta):
    B, T, J, C = x.shape
    xt = jnp.transpose(x, (0, 2, 1, 3))
    ub = U_b.reshape(1, C)
    vb = V_b.reshape(1, C)
    y, stats = pl.pallas_call(
        _phase1_kernel,
        grid=(B,),
        in_specs=[
            pl.BlockSpec((1, J, T, C), lambda b: (b, 0, 0, 0)),
            pl.BlockSpec((C, C), lambda b: (0, 0)),
            pl.BlockSpec((1, C), lambda b: (0, 0)),
            pl.BlockSpec((C, C), lambda b: (0, 0)),
            pl.BlockSpec((1, C), lambda b: (0, 0)),
        ],
        out_specs=[
            pl.BlockSpec((1, J, T, C), lambda b: (b, 0, 0, 0)),
            pl.BlockSpec((2, T), lambda b: (0, 0)),
        ],
        out_shape=[
            jax.ShapeDtypeStruct((B, J, T, C), jnp.float32),
            jax.ShapeDtypeStruct((2, T), jnp.float32),
        ],
    )(xt, U_w, ub, V_w, vb)
    n = B * J * C
    mean = stats[0] / n
    var = stats[1] / n - mean * mean
    scale = bn_gamma * jax.lax.rsqrt(var + _EPS)
    shift = bn_beta - mean * scale
    out_t = pl.pallas_call(
        _phase2_kernel,
        grid=(B,),
        in_specs=[
            pl.BlockSpec((1, J, T, C), lambda b: (b, 0, 0, 0)),
            pl.BlockSpec((1, J, T, C), lambda b: (b, 0, 0, 0)),
            pl.BlockSpec((1, 1, T, 1), lambda b: (0, 0, 0, 0)),
            pl.BlockSpec((1, 1, T, 1), lambda b: (0, 0, 0, 0)),
        ],
        out_specs=pl.BlockSpec((1, J, T, C), lambda b: (b, 0, 0, 0)),
        out_shape=jax.ShapeDtypeStruct((B, J, T, C), jnp.float32),
    )(xt, y, scale.reshape(1, 1, T, 1), shift.reshape(1, 1, T, 1))
    return jnp.transpose(out_t, (0, 2, 1, 3))
